# baseline (speedup 1.0000x reference)
import functools

import jax
import jax.numpy as jnp
from jax import lax
from jax.experimental import pallas as pl
from jax.experimental.pallas import tpu as pltpu
from jax.experimental.pallas import tpu_sc as plsc

D = 128
HEADS = 4
C = 32
AUGD = 144
NC = 2
NS = 16
NW = NC * NS
CHUNK = 48
GRP = 16


def _projections(x, W_l, b_l, W_r, b_r):
    n = x.shape[0]
    blk = 1000

    def body(x_ref, wl_ref, bl_ref, wr_ref, br_ref, xl_ref, xr_ref):
        xb = x_ref[...]
        xl_ref[...] = jnp.dot(xb, wl_ref[...],
                              preferred_element_type=jnp.float32) + bl_ref[...]
        xr_ref[...] = jnp.dot(xb, wr_ref[...],
                              preferred_element_type=jnp.float32) + br_ref[...]

    return pl.pallas_call(
        body,
        grid=(n // blk,),
        in_specs=[
            pl.BlockSpec((blk, D), lambda i: (i, 0)),
            pl.BlockSpec((D, D), lambda i: (0, 0)),
            pl.BlockSpec((1, D), lambda i: (0, 0)),
            pl.BlockSpec((D, D), lambda i: (0, 0)),
            pl.BlockSpec((1, D), lambda i: (0, 0)),
        ],
        out_specs=[pl.BlockSpec((blk, D), lambda i: (i, 0)),
                   pl.BlockSpec((blk, D), lambda i: (i, 0))],
        out_shape=[jax.ShapeDtypeStruct((n, D), jnp.float32),
                   jax.ShapeDtypeStruct((n, D), jnp.float32)],
    )(x, W_l, b_l.reshape(1, D), W_r, b_r.reshape(1, D))


def _sc_aggregate(xl, xr, srcf, dstf, attf, n):
    per_tile = srcf.shape[0] // NW
    nchunks = per_tile // CHUNK
    acc_rows = n + 8
    rows_per_tile = (n // NS) // 8 * 8
    mesh = plsc.VectorSubcoreMesh(core_axis_name="c", subcore_axis_name="s",
                                  num_cores=NC, num_subcores=NS)

    @functools.partial(
        pl.kernel,
        out_type=jax.ShapeDtypeStruct((NC * n, AUGD), jnp.float32),
        mesh=mesh,
        compiler_params=pltpu.CompilerParams(needs_layout_passes=False,
                                             use_tc_tiling_on_sc=False),
        scratch_types=[
            pltpu.VMEM((CHUNK,), jnp.int32),
            pltpu.VMEM((CHUNK,), jnp.int32),
            pltpu.VMEM((CHUNK,), jnp.int32),
            pltpu.VMEM((CHUNK,), jnp.int32),
            pltpu.VMEM((CHUNK, D), jnp.float32),
            pltpu.VMEM((CHUNK, D), jnp.float32),
            pltpu.VMEM((CHUNK, D), jnp.float32),
            pltpu.VMEM((CHUNK, D), jnp.float32),
            pltpu.VMEM((CHUNK, AUGD), jnp.float32),
            pltpu.VMEM((D,), jnp.float32),
            pltpu.VMEM_SHARED((acc_rows, AUGD), jnp.float32),
            pltpu.SemaphoreType.DMA,
            pltpu.SemaphoreType.DMA,
            pltpu.SemaphoreType.DMA,
            pltpu.SemaphoreType.DMA,
        ],
    )
    def k(xl_hbm, xr_hbm, src_hbm, dst_hbm, att_hbm, out_hbm,
          sidx0, sidx1, didx0, didx1, xlb0, xlb1, xrb0, xrb1,
          msg, attv, acc, sxl0, sxl1, sxr0, sxr1):
        cid = lax.axis_index("c")
        sid = lax.axis_index("s")
        wid = cid * NS + sid

        pltpu.sync_copy(att_hbm, attv)

        zeros16 = jnp.zeros((GRP,), jnp.float32)

        def zrow(r, carry):
            for k9 in range(AUGD // 16):
                msg[r, pl.ds(k9 * 16, 16)] = zeros16
            return carry

        lax.fori_loop(0, CHUNK, zrow, 0)

        rbase = sid * rows_per_tile
        ncopy = rows_per_tile // CHUNK
        rem = rows_per_tile - ncopy * CHUNK
        for j in range(ncopy):
            pltpu.sync_copy(msg.at[pl.ds(0, CHUNK)],
                            acc.at[pl.ds(rbase + j * CHUNK, CHUNK)])
        if rem:
            pltpu.sync_copy(msg.at[pl.ds(0, rem)],
                            acc.at[pl.ds(rbase + ncopy * CHUNK, rem)])

        tail = n - NS * rows_per_tile

        @pl.when(sid == 0)
        def _():
            pltpu.sync_copy(msg.at[pl.ds(0, tail + 8)],
                            acc.at[pl.ds(NS * rows_per_tile, tail + 8)])

        plsc.subcore_barrier()

        iota16 = lax.iota(jnp.int32, GRP)
        ebase = wid * per_tile

        def issue(t, sidx_b, didx_b, xlb_b, xrb_b, sem_xl, sem_xr):
            base = ebase + t * CHUNK
            pltpu.sync_copy(src_hbm.at[pl.ds(base, CHUNK)], sidx_b)
            pltpu.sync_copy(dst_hbm.at[pl.ds(base, CHUNK)], didx_b)
            pltpu.async_copy(xl_hbm.at[sidx_b], xlb_b, sem_xl)
            pltpu.async_copy(xr_hbm.at[didx_b], xrb_b, sem_xr)

        def process(sidx_b, didx_b, xlb_b, xrb_b, sem_xl, sem_xr):
            pltpu.make_async_copy(xl_hbm.at[sidx_b], xlb_b, sem_xl).wait()
            pltpu.make_async_copy(xr_hbm.at[didx_b], xrb_b, sem_xr).wait()

            def group_body(g, gcarry):
                rows = g * GRP + iota16
                eh = [jnp.zeros((GRP,), jnp.float32) for _ in range(HEADS)]
                for kk in range(D // 16):
                    av = attv[pl.ds(kk * 16, 16)]
                    for j in range(16):
                        c = kk * 16 + j
                        col = jnp.full((GRP,), c, jnp.int32)
                        a = plsc.load_gather(xlb_b, [rows, col])
                        b = plsc.load_gather(xrb_b, [rows, col])
                        z = a + b
                        m = jnp.where(z > 0, z, 0.2 * z)
                        eh[c // C] = eh[c // C] + m * av[j]
                e_vecs = [jnp.exp(eh[h]) for h in range(HEADS)]
                for h in range(HEADS):
                    ev = e_vecs[h]

                    def mblk(kk, mcarry, ev=ev):
                        base = kk * 16
                        for j in range(16):
                            col = jnp.broadcast_to(base + j, (GRP,))
                            a = plsc.load_gather(xlb_b, [rows, col])
                            plsc.store_scatter(msg, [rows, col], a * ev)
                        return mcarry

                    lax.fori_loop(2 * h, 2 * h + 2, mblk, 0)
                for h in range(HEADS):
                    colh = jnp.full((GRP,), D + h, jnp.int32)
                    plsc.store_scatter(msg, [rows, colh], e_vecs[h])
                return gcarry

            lax.fori_loop(0, CHUNK // GRP, group_body, 0)
            pltpu.sync_copy(msg, acc.at[didx_b], add=True)

        npairs = nchunks // 2
        issue(0, sidx0, didx0, xlb0, xrb0, sxl0, sxr0)

        def pair_body(i, carry):
            t0 = 2 * i
            issue(t0 + 1, sidx1, didx1, xlb1, xrb1, sxl1, sxr1)
            process(sidx0, didx0, xlb0, xrb0, sxl0, sxr0)

            @pl.when(i + 1 < npairs)
            def _():
                issue(t0 + 2, sidx0, didx0, xlb0, xrb0, sxl0, sxr0)

            process(sidx1, didx1, xlb1, xrb1, sxl1, sxr1)
            return carry

        lax.fori_loop(0, npairs, pair_body, 0)
        plsc.subcore_barrier()

        obase = cid * n + rbase
        for j in range(ncopy):
            pltpu.sync_copy(acc.at[pl.ds(rbase + j * CHUNK, CHUNK)],
                            out_hbm.at[pl.ds(obase + j * CHUNK, CHUNK)])
        if rem:
            pltpu.sync_copy(acc.at[pl.ds(rbase + ncopy * CHUNK, rem)],
                            out_hbm.at[pl.ds(obase + ncopy * CHUNK, rem)])

        @pl.when(sid == 0)
        def _():
            pltpu.sync_copy(acc.at[pl.ds(NS * rows_per_tile, tail)],
                            out_hbm.at[pl.ds(cid * n + NS * rows_per_tile,
                                             tail)])

    return k(xl, xr, srcf, dstf, attf)


def _finalize(acc2, x, bias, bn_gamma, bn_beta):
    n = x.shape[0]

    def body(acc_ref, x_ref, b_ref, g_ref, be_ref, o_ref):
        s = acc_ref[0] + acc_ref[1]
        cols = []
        for h in range(HEADS):
            den = s[:, D + h:D + h + 1] + 1e-16
            cols.append(s[:, h * C:(h + 1) * C] / den)
        pre = jnp.concatenate(cols, axis=1) + b_ref[...]
        mean = jnp.mean(pre, axis=0, keepdims=True)
        var = jnp.mean((pre - mean) ** 2, axis=0, keepdims=True)
        y = (pre - mean) * lax.rsqrt(var + 1e-5) * g_ref[...] + be_ref[...]
        y = y + x_ref[...]
        o_ref[...] = jnp.where(y > 0, y, jnp.exp(jnp.minimum(y, 0.0)) - 1.0)

    return pl.pallas_call(
        body,
        in_specs=[
            pl.BlockSpec((2, n, AUGD), lambda: (0, 0, 0)),
            pl.BlockSpec((n, D), lambda: (0, 0)),
            pl.BlockSpec((1, D), lambda: (0, 0)),
            pl.BlockSpec((1, D), lambda: (0, 0)),
            pl.BlockSpec((1, D), lambda: (0, 0)),
        ],
        out_specs=pl.BlockSpec((n, D), lambda: (0, 0)),
        out_shape=jax.ShapeDtypeStruct((n, D), jnp.float32),
    )(acc2.reshape(2, n, AUGD), x, bias.reshape(1, D),
      bn_gamma.reshape(1, D), bn_beta.reshape(1, D))


def kernel(x, edge_index, W_l, b_l, W_r, b_r, att, bias, bn_gamma, bn_beta):
    n = x.shape[0]
    e = edge_index.shape[1]
    src = edge_index[0].astype(jnp.int32)
    dst = edge_index[1].astype(jnp.int32)
    loops = jnp.arange(n, dtype=jnp.int32)
    m = e + n
    nch = -(-m // (NW * CHUNK))
    nch += nch % 2
    per_tile = nch * CHUNK
    pad = per_tile * NW - m
    srcf = jnp.concatenate([src, loops, jnp.zeros((pad,), jnp.int32)])
    dstf = jnp.concatenate([dst, loops, jnp.full((pad,), n, jnp.int32)])
    attf = att.reshape(D)

    xl, xr = _projections(x, W_l, b_l, W_r, b_r)
    acc2 = _sc_aggregate(xl, xr, srcf, dstf, attf, n)
    return _finalize(acc2, x, bias, bn_gamma, bn_beta)


# speedup vs baseline: 1.1879x; 1.1879x over previous
import functools

import jax
import jax.numpy as jnp
from jax import lax
from jax.experimental import pallas as pl
from jax.experimental.pallas import tpu as pltpu
from jax.experimental.pallas import tpu_sc as plsc

D = 128
HEADS = 4
C = 32
AUGD = 144
NC = 2
NS = 16
NW = NC * NS
CHUNK = 48
GRP = 16


def _projections(x, W_l, b_l, W_r, b_r):
    n = x.shape[0]
    blk = 1000

    def body(x_ref, wl_ref, bl_ref, wr_ref, br_ref, xl_ref, xr_ref):
        xb = x_ref[...]
        xl_ref[...] = jnp.dot(xb, wl_ref[...],
                              preferred_element_type=jnp.float32) + bl_ref[...]
        xr_ref[...] = jnp.dot(xb, wr_ref[...],
                              preferred_element_type=jnp.float32) + br_ref[...]

    return pl.pallas_call(
        body,
        grid=(n // blk,),
        in_specs=[
            pl.BlockSpec((blk, D), lambda i: (i, 0)),
            pl.BlockSpec((D, D), lambda i: (0, 0)),
            pl.BlockSpec((1, D), lambda i: (0, 0)),
            pl.BlockSpec((D, D), lambda i: (0, 0)),
            pl.BlockSpec((1, D), lambda i: (0, 0)),
        ],
        out_specs=[pl.BlockSpec((blk, D), lambda i: (i, 0)),
                   pl.BlockSpec((blk, D), lambda i: (i, 0))],
        out_shape=[jax.ShapeDtypeStruct((n, D), jnp.float32),
                   jax.ShapeDtypeStruct((n, D), jnp.float32)],
    )(x, W_l, b_l.reshape(1, D), W_r, b_r.reshape(1, D))


def _sc_aggregate(xl, xr, srcf, dstf, attf, n):
    per_tile = srcf.shape[0] // NW
    nchunks = per_tile // CHUNK
    acc_rows = n + 8
    rows_per_tile = (n // NS) // 8 * 8
    mesh = plsc.VectorSubcoreMesh(core_axis_name="c", subcore_axis_name="s",
                                  num_cores=NC, num_subcores=NS)

    @functools.partial(
        pl.kernel,
        out_type=jax.ShapeDtypeStruct((NC * n, AUGD), jnp.float32),
        mesh=mesh,
        compiler_params=pltpu.CompilerParams(needs_layout_passes=False,
                                             use_tc_tiling_on_sc=False),
        scratch_types=[
            pltpu.VMEM((CHUNK,), jnp.int32),
            pltpu.VMEM((CHUNK,), jnp.int32),
            pltpu.VMEM((CHUNK,), jnp.int32),
            pltpu.VMEM((CHUNK,), jnp.int32),
            pltpu.VMEM((CHUNK, D), jnp.float32),
            pltpu.VMEM((CHUNK, D), jnp.float32),
            pltpu.VMEM((CHUNK, D), jnp.float32),
            pltpu.VMEM((CHUNK, D), jnp.float32),
            pltpu.VMEM((CHUNK, AUGD), jnp.float32),
            pltpu.VMEM(((CHUNK // GRP) * HEADS * GRP,), jnp.float32),
            pltpu.VMEM((D,), jnp.float32),
            pltpu.VMEM_SHARED((acc_rows, AUGD), jnp.float32),
            pltpu.SemaphoreType.DMA,
            pltpu.SemaphoreType.DMA,
            pltpu.SemaphoreType.DMA,
            pltpu.SemaphoreType.DMA,
        ],
    )
    def k(xl_hbm, xr_hbm, src_hbm, dst_hbm, att_hbm, out_hbm,
          sidx0, sidx1, didx0, didx1, xlb0, xlb1, xrb0, xrb1,
          msg, ebuf, attv, acc, sxl0, sxl1, sxr0, sxr1):
        cid = lax.axis_index("c")
        sid = lax.axis_index("s")
        wid = cid * NS + sid

        pltpu.sync_copy(att_hbm, attv)

        zeros16 = jnp.zeros((GRP,), jnp.float32)

        def zrow(r, carry):
            for k9 in range(AUGD // 16):
                msg[r, pl.ds(k9 * 16, 16)] = zeros16
            return carry

        lax.fori_loop(0, CHUNK, zrow, 0)

        rbase = sid * rows_per_tile
        ncopy = rows_per_tile // CHUNK
        rem = rows_per_tile - ncopy * CHUNK
        for j in range(ncopy):
            pltpu.sync_copy(msg.at[pl.ds(0, CHUNK)],
                            acc.at[pl.ds(rbase + j * CHUNK, CHUNK)])
        if rem:
            pltpu.sync_copy(msg.at[pl.ds(0, rem)],
                            acc.at[pl.ds(rbase + ncopy * CHUNK, rem)])

        tail = n - NS * rows_per_tile

        @pl.when(sid == 0)
        def _():
            pltpu.sync_copy(msg.at[pl.ds(0, tail + 8)],
                            acc.at[pl.ds(NS * rows_per_tile, tail + 8)])

        plsc.subcore_barrier()

        iota16 = lax.iota(jnp.int32, GRP)
        ebase = wid * per_tile

        def issue(t, sidx_b, didx_b, xlb_b, xrb_b, sem_xl, sem_xr):
            base = ebase + t * CHUNK
            pltpu.sync_copy(src_hbm.at[pl.ds(base, CHUNK)], sidx_b)
            pltpu.sync_copy(dst_hbm.at[pl.ds(base, CHUNK)], didx_b)
            pltpu.async_copy(xl_hbm.at[sidx_b], xlb_b, sem_xl)
            pltpu.async_copy(xr_hbm.at[didx_b], xrb_b, sem_xr)

        def process(sidx_b, didx_b, xlb_b, xrb_b, sem_xl, sem_xr):
            pltpu.make_async_copy(xl_hbm.at[sidx_b], xlb_b, sem_xl).wait()
            pltpu.make_async_copy(xr_hbm.at[didx_b], xrb_b, sem_xr).wait()

            def e_body(g, gcarry):
                rows = g * GRP + iota16
                eh = [jnp.zeros((GRP,), jnp.float32) for _ in range(HEADS)]
                for kk in range(D // 16):
                    av = attv[pl.ds(kk * 16, 16)]
                    for j in range(16):
                        c = kk * 16 + j
                        col = jnp.full((GRP,), c, jnp.int32)
                        a = plsc.load_gather(xlb_b, [rows, col])
                        b = plsc.load_gather(xrb_b, [rows, col])
                        z = a + b
                        m = jnp.where(z > 0, z, 0.2 * z)
                        eh[c // C] = eh[c // C] + m * av[j]
                for h in range(HEADS):
                    ebuf[pl.ds((g * HEADS + h) * GRP, GRP)] = jnp.exp(eh[h])
                return gcarry

            lax.fori_loop(0, CHUNK // GRP, e_body, 0)

            def m_body(g, gcarry):
                rows = g * GRP + iota16
                evs = [ebuf[pl.ds((g * HEADS + h) * GRP, GRP)]
                       for h in range(HEADS)]
                for c in range(D):
                    col = jnp.full((GRP,), c, jnp.int32)
                    a = plsc.load_gather(xlb_b, [rows, col])
                    plsc.store_scatter(msg, [rows, col], a * evs[c // C])
                for h in range(HEADS):
                    colh = jnp.full((GRP,), D + h, jnp.int32)
                    plsc.store_scatter(msg, [rows, colh], evs[h])
                return gcarry

            lax.fori_loop(0, CHUNK // GRP, m_body, 0)
            pltpu.sync_copy(msg, acc.at[didx_b], add=True)

        npairs = nchunks // 2
        issue(0, sidx0, didx0, xlb0, xrb0, sxl0, sxr0)

        def pair_body(i, carry):
            t0 = 2 * i
            issue(t0 + 1, sidx1, didx1, xlb1, xrb1, sxl1, sxr1)
            process(sidx0, didx0, xlb0, xrb0, sxl0, sxr0)

            @pl.when(i + 1 < npairs)
            def _():
                issue(t0 + 2, sidx0, didx0, xlb0, xrb0, sxl0, sxr0)

            process(sidx1, didx1, xlb1, xrb1, sxl1, sxr1)
            return carry

        lax.fori_loop(0, npairs, pair_body, 0)
        plsc.subcore_barrier()

        obase = cid * n + rbase
        for j in range(ncopy):
            pltpu.sync_copy(acc.at[pl.ds(rbase + j * CHUNK, CHUNK)],
                            out_hbm.at[pl.ds(obase + j * CHUNK, CHUNK)])
        if rem:
            pltpu.sync_copy(acc.at[pl.ds(rbase + ncopy * CHUNK, rem)],
                            out_hbm.at[pl.ds(obase + ncopy * CHUNK, rem)])

        @pl.when(sid == 0)
        def _():
            pltpu.sync_copy(acc.at[pl.ds(NS * rows_per_tile, tail)],
                            out_hbm.at[pl.ds(cid * n + NS * rows_per_tile,
                                             tail)])

    return k(xl, xr, srcf, dstf, attf)


def _finalize(acc2, x, bias, bn_gamma, bn_beta):
    n = x.shape[0]

    def body(acc_ref, x_ref, b_ref, g_ref, be_ref, o_ref):
        s = acc_ref[0] + acc_ref[1]
        cols = []
        for h in range(HEADS):
            den = s[:, D + h:D + h + 1] + 1e-16
            cols.append(s[:, h * C:(h + 1) * C] / den)
        pre = jnp.concatenate(cols, axis=1) + b_ref[...]
        mean = jnp.mean(pre, axis=0, keepdims=True)
        var = jnp.mean((pre - mean) ** 2, axis=0, keepdims=True)
        y = (pre - mean) * lax.rsqrt(var + 1e-5) * g_ref[...] + be_ref[...]
        y = y + x_ref[...]
        o_ref[...] = jnp.where(y > 0, y, jnp.exp(jnp.minimum(y, 0.0)) - 1.0)

    return pl.pallas_call(
        body,
        in_specs=[
            pl.BlockSpec((2, n, AUGD), lambda: (0, 0, 0)),
            pl.BlockSpec((n, D), lambda: (0, 0)),
            pl.BlockSpec((1, D), lambda: (0, 0)),
            pl.BlockSpec((1, D), lambda: (0, 0)),
            pl.BlockSpec((1, D), lambda: (0, 0)),
        ],
        out_specs=pl.BlockSpec((n, D), lambda: (0, 0)),
        out_shape=jax.ShapeDtypeStruct((n, D), jnp.float32),
    )(acc2.reshape(2, n, AUGD), x, bias.reshape(1, D),
      bn_gamma.reshape(1, D), bn_beta.reshape(1, D))


def kernel(x, edge_index, W_l, b_l, W_r, b_r, att, bias, bn_gamma, bn_beta):
    n = x.shape[0]
    e = edge_index.shape[1]
    src = edge_index[0].astype(jnp.int32)
    dst = edge_index[1].astype(jnp.int32)
    loops = jnp.arange(n, dtype=jnp.int32)
    m = e + n
    nch = -(-m // (NW * CHUNK))
    nch += nch % 2
    per_tile = nch * CHUNK
    pad = per_tile * NW - m
    srcf = jnp.concatenate([src, loops, jnp.zeros((pad,), jnp.int32)])
    dstf = jnp.concatenate([dst, loops, jnp.full((pad,), n, jnp.int32)])
    attf = att.reshape(D)

    xl, xr = _projections(x, W_l, b_l, W_r, b_r)
    acc2 = _sc_aggregate(xl, xr, srcf, dstf, attf, n)
    return _finalize(acc2, x, bias, bn_gamma, bn_beta)


# speedup vs baseline: 1.6218x; 1.3652x over previous
import functools

import jax
import jax.numpy as jnp
from jax import lax
from jax.experimental import pallas as pl
from jax.experimental.pallas import tpu as pltpu
from jax.experimental.pallas import tpu_sc as plsc

D = 128
HEADS = 4
C = 32
AUGD = 144
NC = 2
NS = 16
NW = NC * NS
CHUNK = 48
GRP = 16


def _projections(x, W_l, b_l, W_r, b_r):
    n = x.shape[0]
    blk = 1000

    def body(x_ref, wl_ref, bl_ref, wr_ref, br_ref, xl_ref, xr_ref):
        xb = x_ref[...]
        xl_ref[...] = jnp.dot(xb, wl_ref[...],
                              preferred_element_type=jnp.float32) + bl_ref[...]
        xr_ref[...] = jnp.dot(xb, wr_ref[...],
                              preferred_element_type=jnp.float32) + br_ref[...]

    return pl.pallas_call(
        body,
        grid=(n // blk,),
        in_specs=[
            pl.BlockSpec((blk, D), lambda i: (i, 0)),
            pl.BlockSpec((D, D), lambda i: (0, 0)),
            pl.BlockSpec((1, D), lambda i: (0, 0)),
            pl.BlockSpec((D, D), lambda i: (0, 0)),
            pl.BlockSpec((1, D), lambda i: (0, 0)),
        ],
        out_specs=[pl.BlockSpec((blk, D), lambda i: (i, 0)),
                   pl.BlockSpec((blk, D), lambda i: (i, 0))],
        out_shape=[jax.ShapeDtypeStruct((n, D), jnp.float32),
                   jax.ShapeDtypeStruct((n, D), jnp.float32)],
    )(x, W_l, b_l.reshape(1, D), W_r, b_r.reshape(1, D))


def _lane_bcast(v, lane):
    return lax.gather(
        v, lane[:, None],
        dimension_numbers=lax.GatherDimensionNumbers(
            offset_dims=(), collapsed_slice_dims=(0,), start_index_map=(0,)),
        slice_sizes=(1,),
        mode=lax.GatherScatterMode.PROMISE_IN_BOUNDS)


def _sc_aggregate(xl, xr, srcf, dstf, attf, n):
    per_tile = srcf.shape[0] // NW
    nchunks = per_tile // CHUNK
    acc_rows = n + 8
    rows_per_tile = (n // NS) // 8 * 8
    mesh = plsc.VectorSubcoreMesh(core_axis_name="c", subcore_axis_name="s",
                                  num_cores=NC, num_subcores=NS)

    @functools.partial(
        pl.kernel,
        out_type=jax.ShapeDtypeStruct((NC * n, AUGD), jnp.float32),
        mesh=mesh,
        compiler_params=pltpu.CompilerParams(needs_layout_passes=False,
                                             use_tc_tiling_on_sc=False),
        scratch_types=[
            pltpu.VMEM((CHUNK,), jnp.int32),
            pltpu.VMEM((CHUNK,), jnp.int32),
            pltpu.VMEM((CHUNK,), jnp.int32),
            pltpu.VMEM((CHUNK,), jnp.int32),
            pltpu.VMEM((CHUNK, D), jnp.float32),
            pltpu.VMEM((CHUNK, D), jnp.float32),
            pltpu.VMEM((CHUNK, D), jnp.float32),
            pltpu.VMEM((CHUNK, D), jnp.float32),
            pltpu.VMEM((CHUNK, AUGD), jnp.float32),
            pltpu.VMEM((D,), jnp.float32),
            pltpu.VMEM_SHARED((acc_rows, AUGD), jnp.float32),
            pltpu.SemaphoreType.DMA,
            pltpu.SemaphoreType.DMA,
            pltpu.SemaphoreType.DMA,
            pltpu.SemaphoreType.DMA,
        ],
    )
    def k(xl_hbm, xr_hbm, src_hbm, dst_hbm, att_hbm, out_hbm,
          sidx0, sidx1, didx0, didx1, xlb0, xlb1, xrb0, xrb1,
          msg, attv, acc, sxl0, sxl1, sxr0, sxr1):
        cid = lax.axis_index("c")
        sid = lax.axis_index("s")
        wid = cid * NS + sid

        pltpu.sync_copy(att_hbm, attv)

        zeros16 = jnp.zeros((GRP,), jnp.float32)

        def zrow(r, carry):
            for k9 in range(AUGD // 16):
                msg[r, pl.ds(k9 * 16, 16)] = zeros16
            return carry

        lax.fori_loop(0, CHUNK, zrow, 0)

        rbase = sid * rows_per_tile
        ncopy = rows_per_tile // CHUNK
        rem = rows_per_tile - ncopy * CHUNK
        for j in range(ncopy):
            pltpu.sync_copy(msg.at[pl.ds(0, CHUNK)],
                            acc.at[pl.ds(rbase + j * CHUNK, CHUNK)])
        if rem:
            pltpu.sync_copy(msg.at[pl.ds(0, rem)],
                            acc.at[pl.ds(rbase + ncopy * CHUNK, rem)])

        tail = n - NS * rows_per_tile

        @pl.when(sid == 0)
        def _():
            pltpu.sync_copy(msg.at[pl.ds(0, tail + 8)],
                            acc.at[pl.ds(NS * rows_per_tile, tail + 8)])

        plsc.subcore_barrier()

        iota16 = lax.iota(jnp.int32, GRP)
        lane_is = [iota16 == h for h in range(HEADS)]
        ebase = wid * per_tile

        def issue(t, sidx_b, didx_b, xlb_b, xrb_b, sem_xl, sem_xr):
            base = ebase + t * CHUNK
            pltpu.sync_copy(src_hbm.at[pl.ds(base, CHUNK)], sidx_b)
            pltpu.sync_copy(dst_hbm.at[pl.ds(base, CHUNK)], didx_b)
            pltpu.async_copy(xl_hbm.at[sidx_b], xlb_b, sem_xl)
            pltpu.async_copy(xr_hbm.at[didx_b], xrb_b, sem_xr)

        def process(sidx_b, didx_b, xlb_b, xrb_b, sem_xl, sem_xr):
            pltpu.make_async_copy(xl_hbm.at[sidx_b], xlb_b, sem_xl).wait()
            pltpu.make_async_copy(xr_hbm.at[didx_b], xrb_b, sem_xr).wait()

            def e_body(g, gcarry):
                rows = g * GRP + iota16
                eh = [jnp.zeros((GRP,), jnp.float32) for _ in range(HEADS)]
                for kk in range(D // 16):
                    av = attv[pl.ds(kk * 16, 16)]
                    for j in range(16):
                        c = kk * 16 + j
                        col = jnp.full((GRP,), c, jnp.int32)
                        a = plsc.load_gather(xlb_b, [rows, col])
                        b = plsc.load_gather(xrb_b, [rows, col])
                        z = a + b
                        m = jnp.where(z > 0, z, 0.2 * z)
                        eh[c // C] = eh[c // C] + m * av[j]
                e_vecs = [jnp.exp(eh[h]) for h in range(HEADS)]
                for r16 in range(GRP):
                    row = g * GRP + r16
                    lane = jnp.full((GRP,), r16, jnp.int32)
                    es = [_lane_bcast(e_vecs[h], lane) for h in range(HEADS)]
                    for kk in range(D // 16):
                        v = xlb_b[row, pl.ds(kk * 16, 16)]
                        msg[row, pl.ds(kk * 16, 16)] = v * es[kk // 2]
                    aug = jnp.where(lane_is[0], es[0], 0.0)
                    for h in range(1, HEADS):
                        aug = jnp.where(lane_is[h], es[h], aug)
                    msg[row, pl.ds(D, 16)] = aug
                return gcarry

            lax.fori_loop(0, CHUNK // GRP, e_body, 0)
            pltpu.sync_copy(msg, acc.at[didx_b], add=True)

        npairs = nchunks // 2
        issue(0, sidx0, didx0, xlb0, xrb0, sxl0, sxr0)

        def pair_body(i, carry):
            t0 = 2 * i
            issue(t0 + 1, sidx1, didx1, xlb1, xrb1, sxl1, sxr1)
            process(sidx0, didx0, xlb0, xrb0, sxl0, sxr0)

            @pl.when(i + 1 < npairs)
            def _():
                issue(t0 + 2, sidx0, didx0, xlb0, xrb0, sxl0, sxr0)

            process(sidx1, didx1, xlb1, xrb1, sxl1, sxr1)
            return carry

        lax.fori_loop(0, npairs, pair_body, 0)
        plsc.subcore_barrier()

        obase = cid * n + rbase
        for j in range(ncopy):
            pltpu.sync_copy(acc.at[pl.ds(rbase + j * CHUNK, CHUNK)],
                            out_hbm.at[pl.ds(obase + j * CHUNK, CHUNK)])
        if rem:
            pltpu.sync_copy(acc.at[pl.ds(rbase + ncopy * CHUNK, rem)],
                            out_hbm.at[pl.ds(obase + ncopy * CHUNK, rem)])

        @pl.when(sid == 0)
        def _():
            pltpu.sync_copy(acc.at[pl.ds(NS * rows_per_tile, tail)],
                            out_hbm.at[pl.ds(cid * n + NS * rows_per_tile,
                                             tail)])

    return k(xl, xr, srcf, dstf, attf)


def _finalize(acc2, x, bias, bn_gamma, bn_beta):
    n = x.shape[0]

    def body(acc_ref, x_ref, b_ref, g_ref, be_ref, o_ref):
        s = acc_ref[0] + acc_ref[1]
        cols = []
        for h in range(HEADS):
            den = s[:, D + h:D + h + 1] + 1e-16
            cols.append(s[:, h * C:(h + 1) * C] / den)
        pre = jnp.concatenate(cols, axis=1) + b_ref[...]
        mean = jnp.mean(pre, axis=0, keepdims=True)
        var = jnp.mean((pre - mean) ** 2, axis=0, keepdims=True)
        y = (pre - mean) * lax.rsqrt(var + 1e-5) * g_ref[...] + be_ref[...]
        y = y + x_ref[...]
        o_ref[...] = jnp.where(y > 0, y, jnp.exp(jnp.minimum(y, 0.0)) - 1.0)

    return pl.pallas_call(
        body,
        in_specs=[
            pl.BlockSpec((2, n, AUGD), lambda: (0, 0, 0)),
            pl.BlockSpec((n, D), lambda: (0, 0)),
            pl.BlockSpec((1, D), lambda: (0, 0)),
            pl.BlockSpec((1, D), lambda: (0, 0)),
            pl.BlockSpec((1, D), lambda: (0, 0)),
        ],
        out_specs=pl.BlockSpec((n, D), lambda: (0, 0)),
        out_shape=jax.ShapeDtypeStruct((n, D), jnp.float32),
    )(acc2.reshape(2, n, AUGD), x, bias.reshape(1, D),
      bn_gamma.reshape(1, D), bn_beta.reshape(1, D))


def kernel(x, edge_index, W_l, b_l, W_r, b_r, att, bias, bn_gamma, bn_beta):
    n = x.shape[0]
    e = edge_index.shape[1]
    src = edge_index[0].astype(jnp.int32)
    dst = edge_index[1].astype(jnp.int32)
    loops = jnp.arange(n, dtype=jnp.int32)
    m = e + n
    nch = -(-m // (NW * CHUNK))
    nch += nch % 2
    per_tile = nch * CHUNK
    pad = per_tile * NW - m
    srcf = jnp.concatenate([src, loops, jnp.zeros((pad,), jnp.int32)])
    dstf = jnp.concatenate([dst, loops, jnp.full((pad,), n, jnp.int32)])
    attf = att.reshape(D)

    xl, xr = _projections(x, W_l, b_l, W_r, b_r)
    acc2 = _sc_aggregate(xl, xr, srcf, dstf, attf, n)
    return _finalize(acc2, x, bias, bn_gamma, bn_beta)


# speedup vs baseline: 3.5065x; 2.1621x over previous
import functools

import jax
import jax.numpy as jnp
from jax import lax
from jax.experimental import pallas as pl
from jax.experimental.pallas import tpu as pltpu
from jax.experimental.pallas import tpu_sc as plsc

D = 128
HEADS = 4
C = 32
AUGD = 144
NC = 2
NS = 16
NW = NC * NS
CHUNK = 48
GRP = 16


def _projections(x, W_l, b_l, W_r, b_r):
    n = x.shape[0]
    blk = 1000

    def body(x_ref, wl_ref, bl_ref, wr_ref, br_ref, xl_ref, xr_ref):
        xb = x_ref[...]
        xl_ref[...] = jnp.dot(xb, wl_ref[...],
                              preferred_element_type=jnp.float32) + bl_ref[...]
        xr_ref[...] = jnp.dot(xb, wr_ref[...],
                              preferred_element_type=jnp.float32) + br_ref[...]

    return pl.pallas_call(
        body,
        grid=(n // blk,),
        in_specs=[
            pl.BlockSpec((blk, D), lambda i: (i, 0)),
            pl.BlockSpec((D, D), lambda i: (0, 0)),
            pl.BlockSpec((1, D), lambda i: (0, 0)),
            pl.BlockSpec((D, D), lambda i: (0, 0)),
            pl.BlockSpec((1, D), lambda i: (0, 0)),
        ],
        out_specs=[pl.BlockSpec((blk, D), lambda i: (i, 0)),
                   pl.BlockSpec((blk, D), lambda i: (i, 0))],
        out_shape=[jax.ShapeDtypeStruct((n, D), jnp.float32),
                   jax.ShapeDtypeStruct((n, D), jnp.float32)],
    )(x, W_l, b_l.reshape(1, D), W_r, b_r.reshape(1, D))


def _lane_bcast(v, lane):
    return lax.gather(
        v, lane[:, None],
        dimension_numbers=lax.GatherDimensionNumbers(
            offset_dims=(), collapsed_slice_dims=(0,), start_index_map=(0,)),
        slice_sizes=(1,),
        mode=lax.GatherScatterMode.PROMISE_IN_BOUNDS)


def _sc_aggregate(xl, xr, srcf, dstf, attf, n):
    per_tile = srcf.shape[0] // NW
    nchunks = per_tile // CHUNK
    acc_rows = n + 8
    rows_per_tile = (n // NS) // 8 * 8
    mesh = plsc.VectorSubcoreMesh(core_axis_name="c", subcore_axis_name="s",
                                  num_cores=NC, num_subcores=NS)

    @functools.partial(
        pl.kernel,
        out_type=jax.ShapeDtypeStruct((NC * n, AUGD), jnp.float32),
        mesh=mesh,
        compiler_params=pltpu.CompilerParams(needs_layout_passes=False,
                                             use_tc_tiling_on_sc=False),
        scratch_types=[
            pltpu.VMEM((CHUNK,), jnp.int32),
            pltpu.VMEM((CHUNK,), jnp.int32),
            pltpu.VMEM((CHUNK,), jnp.int32),
            pltpu.VMEM((CHUNK,), jnp.int32),
            pltpu.VMEM((CHUNK, D), jnp.float32),
            pltpu.VMEM((CHUNK, D), jnp.float32),
            pltpu.VMEM((CHUNK, D), jnp.float32),
            pltpu.VMEM((CHUNK, D), jnp.float32),
            pltpu.VMEM((CHUNK, AUGD), jnp.float32),
            pltpu.VMEM((D,), jnp.float32),
            pltpu.VMEM_SHARED((acc_rows, AUGD), jnp.float32),
            pltpu.SemaphoreType.DMA,
            pltpu.SemaphoreType.DMA,
            pltpu.SemaphoreType.DMA,
            pltpu.SemaphoreType.DMA,
        ],
    )
    def k(xl_hbm, xr_hbm, src_hbm, dst_hbm, att_hbm, out_hbm,
          sidx0, sidx1, didx0, didx1, xlb0, xlb1, xrb0, xrb1,
          msg, attv, acc, sxl0, sxl1, sxr0, sxr1):
        cid = lax.axis_index("c")
        sid = lax.axis_index("s")
        wid = cid * NS + sid

        pltpu.sync_copy(att_hbm, attv)

        zeros16 = jnp.zeros((GRP,), jnp.float32)

        def zrow(r, carry):
            for k9 in range(AUGD // 16):
                msg[r, pl.ds(k9 * 16, 16)] = zeros16
            return carry

        lax.fori_loop(0, CHUNK, zrow, 0)

        rbase = sid * rows_per_tile
        ncopy = rows_per_tile // CHUNK
        rem = rows_per_tile - ncopy * CHUNK
        for j in range(ncopy):
            pltpu.sync_copy(msg.at[pl.ds(0, CHUNK)],
                            acc.at[pl.ds(rbase + j * CHUNK, CHUNK)])
        if rem:
            pltpu.sync_copy(msg.at[pl.ds(0, rem)],
                            acc.at[pl.ds(rbase + ncopy * CHUNK, rem)])

        tail = n - NS * rows_per_tile

        @pl.when(sid == 0)
        def _():
            pltpu.sync_copy(msg.at[pl.ds(0, tail + 8)],
                            acc.at[pl.ds(NS * rows_per_tile, tail + 8)])

        plsc.subcore_barrier()

        iota16 = lax.iota(jnp.int32, GRP)
        lane_is = [iota16 == h for h in range(HEADS)]
        xor_idx = [jnp.bitwise_xor(iota16, 1 << b) for b in range(4)]
        att_vregs = [attv[pl.ds(kk * 16, 16)] for kk in range(D // 16)]
        ebase = wid * per_tile

        def issue(t, sidx_b, didx_b, xlb_b, xrb_b, sem_xl, sem_xr):
            base = ebase + t * CHUNK
            pltpu.sync_copy(src_hbm.at[pl.ds(base, CHUNK)], sidx_b)
            pltpu.sync_copy(dst_hbm.at[pl.ds(base, CHUNK)], didx_b)
            pltpu.async_copy(xl_hbm.at[sidx_b], xlb_b, sem_xl)
            pltpu.async_copy(xr_hbm.at[didx_b], xrb_b, sem_xr)

        def process(sidx_b, didx_b, xlb_b, xrb_b, sem_xl, sem_xr):
            pltpu.make_async_copy(xl_hbm.at[sidx_b], xlb_b, sem_xl).wait()
            pltpu.make_async_copy(xr_hbm.at[didx_b], xrb_b, sem_xr).wait()

            def e_body(g, gcarry):
                for r16 in range(GRP):
                    row = g * GRP + r16
                    xl_vals = []
                    e_b = []
                    for h in range(HEADS):
                        s = None
                        for q in range(2):
                            kk = 2 * h + q
                            aL = xlb_b[row, pl.ds(kk * 16, 16)]
                            aR = xrb_b[row, pl.ds(kk * 16, 16)]
                            xl_vals.append(aL)
                            z = aL + aR
                            m = jnp.where(z > 0, z, 0.2 * z)
                            t = m * att_vregs[kk]
                            s = t if s is None else s + t
                        for bidx in range(4):
                            s = s + _lane_bcast(s, xor_idx[bidx])
                        e_b.append(jnp.exp(s))
                    for kk in range(D // 16):
                        msg[row, pl.ds(kk * 16, 16)] = (
                            xl_vals[kk] * e_b[kk // 2])
                    aug = jnp.where(lane_is[0], e_b[0], 0.0)
                    for h in range(1, HEADS):
                        aug = jnp.where(lane_is[h], e_b[h], aug)
                    msg[row, pl.ds(D, 16)] = aug
                return gcarry

            lax.fori_loop(0, CHUNK // GRP, e_body, 0)
            pltpu.sync_copy(msg, acc.at[didx_b], add=True)

        npairs = nchunks // 2
        issue(0, sidx0, didx0, xlb0, xrb0, sxl0, sxr0)

        def pair_body(i, carry):
            t0 = 2 * i
            issue(t0 + 1, sidx1, didx1, xlb1, xrb1, sxl1, sxr1)
            process(sidx0, didx0, xlb0, xrb0, sxl0, sxr0)

            @pl.when(i + 1 < npairs)
            def _():
                issue(t0 + 2, sidx0, didx0, xlb0, xrb0, sxl0, sxr0)

            process(sidx1, didx1, xlb1, xrb1, sxl1, sxr1)
            return carry

        lax.fori_loop(0, npairs, pair_body, 0)
        plsc.subcore_barrier()

        obase = cid * n + rbase
        for j in range(ncopy):
            pltpu.sync_copy(acc.at[pl.ds(rbase + j * CHUNK, CHUNK)],
                            out_hbm.at[pl.ds(obase + j * CHUNK, CHUNK)])
        if rem:
            pltpu.sync_copy(acc.at[pl.ds(rbase + ncopy * CHUNK, rem)],
                            out_hbm.at[pl.ds(obase + ncopy * CHUNK, rem)])

        @pl.when(sid == 0)
        def _():
            pltpu.sync_copy(acc.at[pl.ds(NS * rows_per_tile, tail)],
                            out_hbm.at[pl.ds(cid * n + NS * rows_per_tile,
                                             tail)])

    return k(xl, xr, srcf, dstf, attf)


def _finalize(acc2, x, bias, bn_gamma, bn_beta):
    n = x.shape[0]

    def body(acc_ref, x_ref, b_ref, g_ref, be_ref, o_ref):
        s = acc_ref[0] + acc_ref[1]
        cols = []
        for h in range(HEADS):
            den = s[:, D + h:D + h + 1] + 1e-16
            cols.append(s[:, h * C:(h + 1) * C] / den)
        pre = jnp.concatenate(cols, axis=1) + b_ref[...]
        mean = jnp.mean(pre, axis=0, keepdims=True)
        var = jnp.mean((pre - mean) ** 2, axis=0, keepdims=True)
        y = (pre - mean) * lax.rsqrt(var + 1e-5) * g_ref[...] + be_ref[...]
        y = y + x_ref[...]
        o_ref[...] = jnp.where(y > 0, y, jnp.exp(jnp.minimum(y, 0.0)) - 1.0)

    return pl.pallas_call(
        body,
        in_specs=[
            pl.BlockSpec((2, n, AUGD), lambda: (0, 0, 0)),
            pl.BlockSpec((n, D), lambda: (0, 0)),
            pl.BlockSpec((1, D), lambda: (0, 0)),
            pl.BlockSpec((1, D), lambda: (0, 0)),
            pl.BlockSpec((1, D), lambda: (0, 0)),
        ],
        out_specs=pl.BlockSpec((n, D), lambda: (0, 0)),
        out_shape=jax.ShapeDtypeStruct((n, D), jnp.float32),
    )(acc2.reshape(2, n, AUGD), x, bias.reshape(1, D),
      bn_gamma.reshape(1, D), bn_beta.reshape(1, D))


def kernel(x, edge_index, W_l, b_l, W_r, b_r, att, bias, bn_gamma, bn_beta):
    n = x.shape[0]
    e = edge_index.shape[1]
    src = edge_index[0].astype(jnp.int32)
    dst = edge_index[1].astype(jnp.int32)
    loops = jnp.arange(n, dtype=jnp.int32)
    m = e + n
    nch = -(-m // (NW * CHUNK))
    nch += nch % 2
    per_tile = nch * CHUNK
    pad = per_tile * NW - m
    srcf = jnp.concatenate([src, loops, jnp.zeros((pad,), jnp.int32)])
    dstf = jnp.concatenate([dst, loops, jnp.full((pad,), n, jnp.int32)])
    attf = att.reshape(D)

    xl, xr = _projections(x, W_l, b_l, W_r, b_r)
    acc2 = _sc_aggregate(xl, xr, srcf, dstf, attf, n)
    return _finalize(acc2, x, bias, bn_gamma, bn_beta)


# speedup vs baseline: 3.7572x; 1.0715x over previous
import functools

import jax
import jax.numpy as jnp
from jax import lax
from jax.experimental import pallas as pl
from jax.experimental.pallas import tpu as pltpu
from jax.experimental.pallas import tpu_sc as plsc

D = 128
HEADS = 4
C = 32
AUGD = 144
NC = 2
NS = 16
NW = NC * NS
CHUNK = 48
GRP = 16


def _projections(x, W_l, b_l, W_r, b_r):
    n = x.shape[0]
    blk = 1000

    def body(x_ref, wl_ref, bl_ref, wr_ref, br_ref, xl_ref, xr_ref):
        xb = x_ref[...]
        xl_ref[...] = jnp.dot(xb, wl_ref[...],
                              preferred_element_type=jnp.float32) + bl_ref[...]
        xr_ref[...] = jnp.dot(xb, wr_ref[...],
                              preferred_element_type=jnp.float32) + br_ref[...]

    return pl.pallas_call(
        body,
        grid=(n // blk,),
        in_specs=[
            pl.BlockSpec((blk, D), lambda i: (i, 0)),
            pl.BlockSpec((D, D), lambda i: (0, 0)),
            pl.BlockSpec((1, D), lambda i: (0, 0)),
            pl.BlockSpec((D, D), lambda i: (0, 0)),
            pl.BlockSpec((1, D), lambda i: (0, 0)),
        ],
        out_specs=[pl.BlockSpec((blk, D), lambda i: (i, 0)),
                   pl.BlockSpec((blk, D), lambda i: (i, 0))],
        out_shape=[jax.ShapeDtypeStruct((n, D), jnp.float32),
                   jax.ShapeDtypeStruct((n, D), jnp.float32)],
    )(x, W_l, b_l.reshape(1, D), W_r, b_r.reshape(1, D))


def _lane_bcast(v, lane):
    return lax.gather(
        v, lane[:, None],
        dimension_numbers=lax.GatherDimensionNumbers(
            offset_dims=(), collapsed_slice_dims=(0,), start_index_map=(0,)),
        slice_sizes=(1,),
        mode=lax.GatherScatterMode.PROMISE_IN_BOUNDS)


def _sc_aggregate(xl, xr, srcf, dstf, attf, n):
    per_tile = srcf.shape[0] // NW
    nchunks = per_tile // CHUNK
    acc_rows = n + 8
    rows_per_tile = (n // NS) // 8 * 8
    mesh = plsc.VectorSubcoreMesh(core_axis_name="c", subcore_axis_name="s",
                                  num_cores=NC, num_subcores=NS)

    @functools.partial(
        pl.kernel,
        out_type=jax.ShapeDtypeStruct((NC * n, AUGD), jnp.float32),
        mesh=mesh,
        compiler_params=pltpu.CompilerParams(needs_layout_passes=False,
                                             use_tc_tiling_on_sc=False),
        scratch_types=[
            pltpu.VMEM((CHUNK,), jnp.int32),
            pltpu.VMEM((CHUNK,), jnp.int32),
            pltpu.VMEM((CHUNK,), jnp.int32),
            pltpu.VMEM((CHUNK,), jnp.int32),
            pltpu.VMEM((CHUNK, D), jnp.float32),
            pltpu.VMEM((CHUNK, D), jnp.float32),
            pltpu.VMEM((CHUNK, D), jnp.float32),
            pltpu.VMEM((CHUNK, D), jnp.float32),
            pltpu.VMEM((CHUNK, AUGD), jnp.float32),
            pltpu.VMEM((CHUNK, AUGD), jnp.float32),
            pltpu.VMEM((CHUNK,), jnp.int32),
            pltpu.VMEM((CHUNK,), jnp.int32),
            pltpu.VMEM((D,), jnp.float32),
            pltpu.VMEM_SHARED((acc_rows, AUGD), jnp.float32),
            pltpu.SemaphoreType.DMA,
            pltpu.SemaphoreType.DMA,
            pltpu.SemaphoreType.DMA,
            pltpu.SemaphoreType.DMA,
            pltpu.SemaphoreType.DMA,
            pltpu.SemaphoreType.DMA,
        ],
    )
    def k(xl_hbm, xr_hbm, src_hbm, dst_hbm, att_hbm, out_hbm,
          sidx0, sidx1, didx0, didx1, xlb0, xlb1, xrb0, xrb1,
          msg0, msg1, didxs0, didxs1, attv, acc,
          sxl0, sxl1, sxr0, sxr1, ssc0, ssc1):
        msg = msg0
        cid = lax.axis_index("c")
        sid = lax.axis_index("s")
        wid = cid * NS + sid

        pltpu.sync_copy(att_hbm, attv)

        zeros16 = jnp.zeros((GRP,), jnp.float32)

        def zrow(r, carry):
            for k9 in range(AUGD // 16):
                msg0[r, pl.ds(k9 * 16, 16)] = zeros16
                msg1[r, pl.ds(k9 * 16, 16)] = zeros16
            return carry

        lax.fori_loop(0, CHUNK, zrow, 0)

        rbase = sid * rows_per_tile
        ncopy = rows_per_tile // CHUNK
        rem = rows_per_tile - ncopy * CHUNK
        for j in range(ncopy):
            pltpu.sync_copy(msg.at[pl.ds(0, CHUNK)],
                            acc.at[pl.ds(rbase + j * CHUNK, CHUNK)])
        if rem:
            pltpu.sync_copy(msg.at[pl.ds(0, rem)],
                            acc.at[pl.ds(rbase + ncopy * CHUNK, rem)])

        tail = n - NS * rows_per_tile

        @pl.when(sid == 0)
        def _():
            pltpu.sync_copy(msg.at[pl.ds(0, tail + 8)],
                            acc.at[pl.ds(NS * rows_per_tile, tail + 8)])

        plsc.subcore_barrier()

        iota16 = lax.iota(jnp.int32, GRP)
        lane_is = [iota16 == h for h in range(HEADS)]
        xor_idx = [jnp.bitwise_xor(iota16, 1 << b) for b in range(4)]
        att_vregs = [attv[pl.ds(kk * 16, 16)] for kk in range(D // 16)]
        ebase = wid * per_tile

        slot0 = (sidx0, didx0, didxs0, xlb0, xrb0, msg0, sxl0, sxr0, ssc0)
        slot1 = (sidx1, didx1, didxs1, xlb1, xrb1, msg1, sxl1, sxr1, ssc1)

        def issue(t, slot):
            sidx_b, didx_b, _, xlb_b, xrb_b, _, sem_xl, sem_xr, _ = slot
            base = ebase + t * CHUNK
            pltpu.sync_copy(src_hbm.at[pl.ds(base, CHUNK)], sidx_b)
            pltpu.sync_copy(dst_hbm.at[pl.ds(base, CHUNK)], didx_b)
            pltpu.async_copy(xl_hbm.at[sidx_b], xlb_b, sem_xl)
            pltpu.async_copy(xr_hbm.at[didx_b], xrb_b, sem_xr)

        def process(slot, do_wait):
            (sidx_b, didx_b, didxs_b, xlb_b, xrb_b, msg,
             sem_xl, sem_xr, sem_sc) = slot
            pltpu.make_async_copy(xl_hbm.at[sidx_b], xlb_b, sem_xl).wait()
            pltpu.make_async_copy(xr_hbm.at[didx_b], xrb_b, sem_xr).wait()

            @pl.when(do_wait)
            def _():
                pltpu.make_async_copy(msg, acc.at[didxs_b], sem_sc).wait()

            for v in range(CHUNK // 16):
                didxs_b[pl.ds(v * 16, 16)] = didx_b[pl.ds(v * 16, 16)]

            def e_body(g, gcarry):
                for r16 in range(GRP):
                    row = g * GRP + r16
                    xl_vals = []
                    e_b = []
                    for h in range(HEADS):
                        s = None
                        for q in range(2):
                            kk = 2 * h + q
                            aL = xlb_b[row, pl.ds(kk * 16, 16)]
                            aR = xrb_b[row, pl.ds(kk * 16, 16)]
                            xl_vals.append(aL)
                            z = aL + aR
                            m = jnp.where(z > 0, z, 0.2 * z)
                            t = m * att_vregs[kk]
                            s = t if s is None else s + t
                        for bidx in range(4):
                            s = s + _lane_bcast(s, xor_idx[bidx])
                        e_b.append(jnp.exp(s))
                    for kk in range(D // 16):
                        msg[row, pl.ds(kk * 16, 16)] = (
                            xl_vals[kk] * e_b[kk // 2])
                    aug = jnp.where(lane_is[0], e_b[0], 0.0)
                    for h in range(1, HEADS):
                        aug = jnp.where(lane_is[h], e_b[h], aug)
                    msg[row, pl.ds(D, 16)] = aug
                return gcarry

            lax.fori_loop(0, CHUNK // GRP, e_body, 0)
            pltpu.async_copy(msg, acc.at[didxs_b], sem_sc, add=True)

        npairs = nchunks // 2
        issue(0, slot0)

        def pair_body(i, carry):
            t0 = 2 * i
            issue(t0 + 1, slot1)
            process(slot0, i > 0)

            @pl.when(i + 1 < npairs)
            def _():
                issue(t0 + 2, slot0)

            process(slot1, i > 0)
            return carry

        lax.fori_loop(0, npairs, pair_body, 0)
        for slot in (slot0, slot1):
            pltpu.make_async_copy(slot[5], acc.at[slot[2]], slot[8]).wait()
        plsc.subcore_barrier()

        obase = cid * n + rbase
        for j in range(ncopy):
            pltpu.sync_copy(acc.at[pl.ds(rbase + j * CHUNK, CHUNK)],
                            out_hbm.at[pl.ds(obase + j * CHUNK, CHUNK)])
        if rem:
            pltpu.sync_copy(acc.at[pl.ds(rbase + ncopy * CHUNK, rem)],
                            out_hbm.at[pl.ds(obase + ncopy * CHUNK, rem)])

        @pl.when(sid == 0)
        def _():
            pltpu.sync_copy(acc.at[pl.ds(NS * rows_per_tile, tail)],
                            out_hbm.at[pl.ds(cid * n + NS * rows_per_tile,
                                             tail)])

    return k(xl, xr, srcf, dstf, attf)


def _finalize(acc2, x, bias, bn_gamma, bn_beta):
    n = x.shape[0]

    def body(acc_ref, x_ref, b_ref, g_ref, be_ref, o_ref):
        s = acc_ref[0] + acc_ref[1]
        cols = []
        for h in range(HEADS):
            den = s[:, D + h:D + h + 1] + 1e-16
            cols.append(s[:, h * C:(h + 1) * C] / den)
        pre = jnp.concatenate(cols, axis=1) + b_ref[...]
        mean = jnp.mean(pre, axis=0, keepdims=True)
        var = jnp.mean((pre - mean) ** 2, axis=0, keepdims=True)
        y = (pre - mean) * lax.rsqrt(var + 1e-5) * g_ref[...] + be_ref[...]
        y = y + x_ref[...]
        o_ref[...] = jnp.where(y > 0, y, jnp.exp(jnp.minimum(y, 0.0)) - 1.0)

    return pl.pallas_call(
        body,
        in_specs=[
            pl.BlockSpec((2, n, AUGD), lambda: (0, 0, 0)),
            pl.BlockSpec((n, D), lambda: (0, 0)),
            pl.BlockSpec((1, D), lambda: (0, 0)),
            pl.BlockSpec((1, D), lambda: (0, 0)),
            pl.BlockSpec((1, D), lambda: (0, 0)),
        ],
        out_specs=pl.BlockSpec((n, D), lambda: (0, 0)),
        out_shape=jax.ShapeDtypeStruct((n, D), jnp.float32),
    )(acc2.reshape(2, n, AUGD), x, bias.reshape(1, D),
      bn_gamma.reshape(1, D), bn_beta.reshape(1, D))


def kernel(x, edge_index, W_l, b_l, W_r, b_r, att, bias, bn_gamma, bn_beta):
    n = x.shape[0]
    e = edge_index.shape[1]
    src = edge_index[0].astype(jnp.int32)
    dst = edge_index[1].astype(jnp.int32)
    loops = jnp.arange(n, dtype=jnp.int32)
    m = e + n
    nch = -(-m // (NW * CHUNK))
    nch += nch % 2
    per_tile = nch * CHUNK
    pad = per_tile * NW - m
    srcf = jnp.concatenate([src, loops, jnp.zeros((pad,), jnp.int32)])
    dstf = jnp.concatenate([dst, loops, jnp.full((pad,), n, jnp.int32)])
    attf = att.reshape(D)

    xl, xr = _projections(x, W_l, b_l, W_r, b_r)
    acc2 = _sc_aggregate(xl, xr, srcf, dstf, attf, n)
    return _finalize(acc2, x, bias, bn_gamma, bn_beta)
